# Initial kernel scaffold; baseline (speedup 1.0000x reference)
import jax
import jax.numpy as jnp
from jax.experimental import pallas as pl


def kernel(x, edge_index, edge_attr, batch, W0, root0, b0, W1, root1, b1, W2, root2, b2, W3, root3, b3, fc1_w, fc1_b, fc2_w, fc2_b):
    raise NotImplementedError("write your pallas kernel here")


# speedup vs baseline: 1.0234x; 1.0234x over previous
import functools

import jax
import jax.numpy as jnp
from jax import lax
from jax.experimental import pallas as pl
from jax.experimental.pallas import tpu as pltpu
from jax.experimental.pallas import tpu_sc as plsc

N = 10000
E = 160000
KS = 3
NG = 8
NSLOT = 9
NPAIR = 6
E_PAD = 163840
N_ACC = 10240
RSUB = N_ACC // 16
RQ = 128


def _prep_body(ea0_ref, ea1_ref, src_ref, wv_ref, basis_ref, pidx_ref):
    p0 = ea0_ref[...] * (KS - 1.0)
    p1 = ea1_ref[...] * (KS - 1.0)
    i0 = jnp.clip(jnp.floor(p0), 0.0, KS - 2.0)
    i1 = jnp.clip(jnp.floor(p1), 0.0, KS - 2.0)
    f0 = p0 - i0
    f1 = p1 - i1
    i0i = i0.astype(jnp.int32)
    i1i = i1.astype(jnp.int32)
    wv = wv_ref[...]
    src = src_ref[...]
    j = 0
    for b0 in (0, 1):
        w0 = f0 if b0 else 1.0 - f0
        for b1 in (0, 1):
            w1 = f1 if b1 else 1.0 - f1
            basis_ref[j] = w0 * w1 * wv
            j += 1
    pr = (2 * i0i + i1i) * N + src
    pidx_ref[0] = pr
    pidx_ref[1] = pr + 2 * N


def _prep(ea0, ea1, src, wv):
    rows = E_PAD // 128
    return pl.pallas_call(
        _prep_body,
        out_shape=[jax.ShapeDtypeStruct((4, rows, 128), jnp.float32),
                   jax.ShapeDtypeStruct((2, rows, 128), jnp.int32)],
    )(ea0, ea1, src, wv)


def _dense_pair(inp, W, oc, scale, split):
    ic = inp.shape[1]
    R = 1000

    def rl(p):
        return 3 * (p // 2) + p % 2

    if split:
        out_shape = jax.ShapeDtypeStruct((2, NPAIR, N, 256), jnp.float32)
        out_spec = pl.BlockSpec((2, 1, R, 256), lambda i, p: (0, p, i, 0))
    else:
        out_shape = jax.ShapeDtypeStruct((NPAIR, N, 2 * oc), jnp.float32)
        out_spec = pl.BlockSpec((1, R, 2 * oc), lambda i, p: (p, i, 0))

    def body(x_ref, wl_ref, wr_ref, o_ref):
        x = x_ref[...] * scale
        yl = jnp.dot(x, wl_ref[0], preferred_element_type=jnp.float32)
        yr = jnp.dot(x, wr_ref[0], preferred_element_type=jnp.float32)
        if split:
            o_ref[0, 0] = jnp.concatenate([yl[:, :128], yr[:, :128]], axis=1)
            o_ref[1, 0] = jnp.concatenate([yl[:, 128:], yr[:, 128:]], axis=1)
        else:
            o_ref[0] = jnp.concatenate([yl, yr], axis=1)

    return pl.pallas_call(
        body,
        grid=(N // R, NPAIR),
        in_specs=[pl.BlockSpec((R, ic), lambda i, p: (i, 0)),
                  pl.BlockSpec((1, ic, oc), lambda i, p: (rl(p), 0, 0)),
                  pl.BlockSpec((1, ic, oc), lambda i, p: (rl(p) + 1, 0, 0))],
        out_specs=out_spec,
        out_shape=out_shape,
    )(inp, W, W)


def _dense_quad(inp, W, scale):
    ic = inp.shape[1]
    oc = W.shape[2]
    R = 1000

    def rl(p):
        return 3 * (p // 2) + p % 2

    def body(x_ref, w0_ref, w1_ref, w2_ref, w3_ref, o_ref):
        x = x_ref[...] * scale
        ys = [jnp.dot(x, w[0], preferred_element_type=jnp.float32)
              for w in (w0_ref, w1_ref, w2_ref, w3_ref)]
        o_ref[0] = jnp.concatenate(ys, axis=1)

    return pl.pallas_call(
        body,
        grid=(N // R, 4),
        in_specs=[pl.BlockSpec((R, ic), lambda i, p: (i, 0)),
                  pl.BlockSpec((1, ic, oc), lambda i, p: (rl(p), 0, 0)),
                  pl.BlockSpec((1, ic, oc), lambda i, p: (rl(p) + 1, 0, 0)),
                  pl.BlockSpec((1, ic, oc), lambda i, p: (rl(p) + 3, 0, 0)),
                  pl.BlockSpec((1, ic, oc), lambda i, p: (rl(p) + 4, 0, 0))],
        out_specs=pl.BlockSpec((1, R, 4 * oc), lambda i, p: (p, i, 0)),
        out_shape=jax.ShapeDtypeStruct((4, N, 4 * oc), jnp.float32),
    )(inp, W, W, W, W)


def _epi(acc, cnt, x, root, b, oc, concat_halves, scale):
    ic = x.shape[1]
    R = 1000
    aw = 128

    def body(a_ref, c_ref, x_ref, r_ref, b_ref, o_ref):
        c0 = c_ref[0, :, 0] + c_ref[1, :, 0]
        rcp = (1.0 / jnp.maximum(c0, 1.0))[:, None]
        if concat_halves:
            s = jnp.concatenate([a_ref[0], a_ref[1]], axis=1)
        else:
            s = (a_ref[0] + a_ref[1])[:, :oc]
        rt = jnp.dot(x_ref[...] * scale, r_ref[...],
                     preferred_element_type=jnp.float32)
        o_ref[...] = jnp.maximum(s * rcp + rt + b_ref[...], 0.0)

    return pl.pallas_call(
        body,
        grid=(N // R,),
        in_specs=[pl.BlockSpec((2, R, aw), lambda i: (0, i, 0)),
                  pl.BlockSpec((2, R, 128), lambda i: (0, i, 0)),
                  pl.BlockSpec((R, ic), lambda i: (i, 0)),
                  pl.BlockSpec((ic, oc), lambda i: (0, 0)),
                  pl.BlockSpec((1, oc), lambda i: (0, 0))],
        out_specs=pl.BlockSpec((R, oc), lambda i: (i, 0)),
        out_shape=jax.ShapeDtypeStruct((N, oc), jnp.float32),
    )(acc, cnt, x, root, b.reshape(1, oc))


def _final(h, batch2, fc1_w, fc1_b, fc2_w, fc2_b):
    def body(h_ref, b_ref, w1_ref, b1_ref, w2_ref, b2_ref, o_ref):
        bt = b_ref[...]
        gids = lax.broadcasted_iota(jnp.int32, (1, NG), 1)
        mask = (bt == gids).astype(jnp.float32)
        gs = lax.dot_general(mask, h_ref[...], (((0,), (0,)), ((), ())),
                             preferred_element_type=jnp.float32)
        gc = jnp.sum(mask, axis=0)[:, None]
        g = gs / jnp.maximum(gc, 1.0)
        t = jnp.maximum(
            jnp.dot(g, w1_ref[...], preferred_element_type=jnp.float32)
            + b1_ref[...], 0.0)
        o_ref[...] = (jnp.dot(t, w2_ref[...], preferred_element_type=jnp.float32)
                      + b2_ref[...])

    return pl.pallas_call(
        body,
        out_shape=jax.ShapeDtypeStruct((NG, 16), jnp.float32),
    )(h, batch2, fc1_w, fc1_b.reshape(1, -1), fc2_w, fc2_b.reshape(1, -1))


def _sc_layer(table, idxf, dst, basis, O, col_split, quad):
    mesh = plsc.VectorSubcoreMesh(core_axis_name="c", subcore_axis_name="s")
    G = O // 16
    C = 128 if quad else 64
    rpe = 1 if quad else 2
    rw = 4 * O if quad else 2 * O
    epw = E_PAD // 16 if col_split else E_PAD // 32
    nch = epw // C

    @functools.partial(
        pl.kernel, mesh=mesh,
        out_type=jax.ShapeDtypeStruct((2, N_ACC, 128), jnp.float32),
        scratch_types=[
            pltpu.VMEM((rpe * C,), jnp.int32),
            pltpu.VMEM((rpe * C,), jnp.int32),
            pltpu.VMEM((C,), jnp.int32),
            pltpu.VMEM((4 * C,), jnp.float32),
            pltpu.VMEM((rpe * C, rw), jnp.float32),
            pltpu.VMEM((C, 128), jnp.float32),
            pltpu.VMEM_SHARED((N_ACC, 128), jnp.float32),
            pltpu.SemaphoreType.DMA,
        ])
    def k(table_h, idx_h, dst_h, basis_h, out_h,
          idx_v, idx2_v, dst_v, basis_v, rows_v, msg_v, acc, sem):
        c = lax.axis_index("c")
        s = lax.axis_index("s")

        def zb(i, _):
            for g in range(8):
                msg_v[i, pl.ds(g * 16, 16)] = jnp.zeros((16,), jnp.float32)
            return 0
        lax.fori_loop(0, C, zb, 0)
        r0 = s * RSUB
        for q in range(RSUB // C):
            pltpu.sync_copy(msg_v, acc.at[pl.ds(r0 + q * C, C)])
        plsc.subcore_barrier()

        if col_split:
            base0 = s * epw
            off = c * (NPAIR * N)
        else:
            base0 = (s * 2 + c) * epw

        def chunk(i, _):
            base = base0 + i * C
            pltpu.sync_copy(idx_h.at[pl.ds(base * rpe, rpe * C)], idx_v)
            pltpu.sync_copy(dst_h.at[pl.ds(base, C)], dst_v)
            pltpu.sync_copy(basis_h.at[pl.ds(base * 4, 4 * C)], basis_v)
            if col_split:
                for g in range(rpe * C // 16):
                    sl = pl.ds(g * 16, 16)
                    idx2_v[sl] = idx_v[sl] + off
                idx_ref = idx2_v
            else:
                idx_ref = idx_v
            pltpu.async_copy(table_h.at[idx_ref], rows_v, sem).wait()

            def edge4(t, _):
                bv = basis_v[pl.ds(t * 16, 16)]
                for q in range(4):
                    e = t * 4 + q
                    for g in range(G):
                        sl = pl.ds(g * 16, 16)
                        if quad:
                            m = (bv[4 * q] * rows_v[e, pl.ds(g * 16, 16)]
                                 + bv[4 * q + 1] * rows_v[e, pl.ds(O + g * 16, 16)]
                                 + bv[4 * q + 2] * rows_v[e, pl.ds(2 * O + g * 16, 16)]
                                 + bv[4 * q + 3] * rows_v[e, pl.ds(3 * O + g * 16, 16)])
                        else:
                            m = (bv[4 * q] * rows_v[2 * e, sl]
                                 + bv[4 * q + 1] * rows_v[2 * e, pl.ds(O + g * 16, 16)]
                                 + bv[4 * q + 2] * rows_v[2 * e + 1, sl]
                                 + bv[4 * q + 3] * rows_v[2 * e + 1, pl.ds(O + g * 16, 16)])
                        msg_v[e, sl] = m
                return 0
            lax.fori_loop(0, C // 4, edge4, 0)
            pltpu.sync_copy(msg_v, acc.at[dst_v], add=True)
            return 0
        lax.fori_loop(0, nch, chunk, 0)
        plsc.subcore_barrier()
        for q in range(5):
            pltpu.sync_copy(acc.at[pl.ds(r0 + q * RQ, RQ)],
                            out_h.at[c, pl.ds(r0 + q * RQ, RQ)])

    return k(table, idxf, dst, basis)


def _sc_counts(dst, wvec):
    mesh = plsc.VectorSubcoreMesh(core_axis_name="c", subcore_axis_name="s")
    C = 64
    epw = E_PAD // 32
    nch = epw // C

    @functools.partial(
        pl.kernel, mesh=mesh,
        out_type=jax.ShapeDtypeStruct((2, N_ACC, 128), jnp.float32),
        scratch_types=[
            pltpu.VMEM((C,), jnp.int32),
            pltpu.VMEM((C,), jnp.float32),
            pltpu.VMEM((C, 128), jnp.float32),
            pltpu.VMEM_SHARED((N_ACC, 128), jnp.float32),
        ])
    def k(dst_h, w_h, out_h, dst_v, w_v, ones_v, acc):
        c = lax.axis_index("c")
        s = lax.axis_index("s")

        def zb(i, _):
            for g in range(8):
                ones_v[i, pl.ds(g * 16, 16)] = jnp.zeros((16,), jnp.float32)
            return 0
        lax.fori_loop(0, C, zb, 0)
        r0 = s * RSUB
        for q in range(RSUB // C):
            pltpu.sync_copy(ones_v, acc.at[pl.ds(r0 + q * C, C)])
        plsc.subcore_barrier()

        base0 = (s * 2 + c) * epw

        def chunk(i, _):
            base = base0 + i * C
            pltpu.sync_copy(dst_h.at[pl.ds(base, C)], dst_v)
            pltpu.sync_copy(w_h.at[pl.ds(base, C)], w_v)

            def edge16(t, _):
                wv16 = w_v[pl.ds(t * 16, 16)]
                for q in range(16):
                    w = wv16[q]
                    for g in range(8):
                        ones_v[t * 16 + q, pl.ds(g * 16, 16)] = (
                            w * jnp.ones((16,), jnp.float32))
                return 0
            lax.fori_loop(0, C // 16, edge16, 0)
            pltpu.sync_copy(ones_v, acc.at[dst_v], add=True)
            return 0
        lax.fori_loop(0, nch, chunk, 0)
        plsc.subcore_barrier()
        for q in range(5):
            pltpu.sync_copy(acc.at[pl.ds(r0 + q * RQ, RQ)],
                            out_h.at[c, pl.ds(r0 + q * RQ, RQ)])

    return k(dst, wvec)


def kernel(x, edge_index, edge_attr, batch, W0, root0, b0, W1, root1, b1,
           W2, root2, b2, W3, root3, b3, fc1_w, fc1_b, fc2_w, fc2_b):
    src = edge_index[0]
    dst = edge_index[1]
    pad = E_PAD - E
    srcp = jnp.concatenate([src, jnp.zeros((pad,), jnp.int32)])
    dstp = jnp.concatenate([dst, jnp.zeros((pad,), jnp.int32)])
    ea = jnp.concatenate([edge_attr, jnp.zeros((pad, 2), jnp.float32)], axis=0)
    wv = jnp.concatenate([jnp.ones((E,), jnp.float32),
                          jnp.zeros((pad,), jnp.float32)])
    rows = E_PAD // 128
    basis4, pidx2 = _prep(ea[:, 0].reshape(rows, 128),
                          ea[:, 1].reshape(rows, 128),
                          srcp.reshape(rows, 128), wv.reshape(rows, 128))
    basis_f = basis4.reshape(4, E_PAD).T.reshape(-1)
    pair_f = pidx2.reshape(2, E_PAD).T.reshape(-1)
    quad_f = pidx2.reshape(2, E_PAD)[0]

    cnt = _sc_counts(dstp, wv)

    table0 = _dense_pair(x, W0, 256, 1.0, True).reshape(2 * NPAIR * N, 256)
    acc0 = _sc_layer(table0, pair_f, dstp, basis_f, 128, True, False)
    h0 = _epi(acc0, cnt, x, root0, b0, 256, True, 1.0)

    table1 = _dense_pair(h0, W1, 128, 2.0, False).reshape(NPAIR * N, 256)
    acc1 = _sc_layer(table1, pair_f, dstp, basis_f, 128, False, False)
    h1 = _epi(acc1, cnt, h0, root1, b1, 128, False, 2.0)

    table2 = _dense_pair(h1, W2, 64, 2.0, False).reshape(NPAIR * N, 128)
    acc2 = _sc_layer(table2, pair_f, dstp, basis_f, 64, False, False)
    h2 = _epi(acc2, cnt, h1, root2, b2, 64, False, 2.0)

    table3 = _dense_quad(h2, W3, 2.0).reshape(4 * N, 128)
    acc3 = _sc_layer(table3, quad_f, dstp, basis_f, 32, False, True)
    h3 = _epi(acc3, cnt, h2, root3, b3, 32, False, 2.0)

    return _final(h3, batch.reshape(N, 1), fc1_w, fc1_b, fc2_w, fc2_b)


# speedup vs baseline: 1.4334x; 1.4007x over previous
import functools

import jax
import jax.numpy as jnp
from jax import lax
from jax.experimental import pallas as pl
from jax.experimental.pallas import tpu as pltpu
from jax.experimental.pallas import tpu_sc as plsc

N = 10000
E = 160000
KS = 3
NG = 8
NSLOT = 9
NPAIR = 6
E_PAD = 163840
N_ACC = 10240
RSUB = N_ACC // 16
RQ = 128


def _prep_body(ea0_ref, ea1_ref, src_ref, wv_ref, basis_ref, pidx_ref):
    p0 = ea0_ref[...] * (KS - 1.0)
    p1 = ea1_ref[...] * (KS - 1.0)
    i0 = jnp.clip(jnp.floor(p0), 0.0, KS - 2.0)
    i1 = jnp.clip(jnp.floor(p1), 0.0, KS - 2.0)
    f0 = p0 - i0
    f1 = p1 - i1
    i0i = i0.astype(jnp.int32)
    i1i = i1.astype(jnp.int32)
    wv = wv_ref[...]
    src = src_ref[...]
    j = 0
    for b0 in (0, 1):
        w0 = f0 if b0 else 1.0 - f0
        for b1 in (0, 1):
            w1 = f1 if b1 else 1.0 - f1
            basis_ref[j] = w0 * w1 * wv
            j += 1
    pr = (2 * i0i + i1i) * N + src
    pidx_ref[0] = pr
    pidx_ref[1] = pr + 2 * N


def _prep(ea0, ea1, src, wv):
    rows = E_PAD // 128
    return pl.pallas_call(
        _prep_body,
        out_shape=[jax.ShapeDtypeStruct((4, rows, 128), jnp.float32),
                   jax.ShapeDtypeStruct((2, rows, 128), jnp.int32)],
    )(ea0, ea1, src, wv)


def _dense_pair(inp, W, oc, scale, split):
    ic = inp.shape[1]
    R = 1000

    def rl(p):
        return 3 * (p // 2) + p % 2

    if split:
        out_shape = jax.ShapeDtypeStruct((2, NPAIR, N, 256), jnp.float32)
        out_spec = pl.BlockSpec((2, 1, R, 256), lambda i, p: (0, p, i, 0))
    else:
        out_shape = jax.ShapeDtypeStruct((NPAIR, N, 2 * oc), jnp.float32)
        out_spec = pl.BlockSpec((1, R, 2 * oc), lambda i, p: (p, i, 0))

    def body(x_ref, wl_ref, wr_ref, o_ref):
        x = x_ref[...] * scale
        yl = jnp.dot(x, wl_ref[0], preferred_element_type=jnp.float32)
        yr = jnp.dot(x, wr_ref[0], preferred_element_type=jnp.float32)
        if split:
            o_ref[0, 0] = jnp.concatenate([yl[:, :128], yr[:, :128]], axis=1)
            o_ref[1, 0] = jnp.concatenate([yl[:, 128:], yr[:, 128:]], axis=1)
        else:
            o_ref[0] = jnp.concatenate([yl, yr], axis=1)

    return pl.pallas_call(
        body,
        grid=(N // R, NPAIR),
        in_specs=[pl.BlockSpec((R, ic), lambda i, p: (i, 0)),
                  pl.BlockSpec((1, ic, oc), lambda i, p: (rl(p), 0, 0)),
                  pl.BlockSpec((1, ic, oc), lambda i, p: (rl(p) + 1, 0, 0))],
        out_specs=out_spec,
        out_shape=out_shape,
    )(inp, W, W)


def _dense_quad(inp, W, scale):
    ic = inp.shape[1]
    oc = W.shape[2]
    R = 1000

    def rl(p):
        return 3 * (p // 2) + p % 2

    def body(x_ref, w0_ref, w1_ref, w2_ref, w3_ref, o_ref):
        x = x_ref[...] * scale
        ys = [jnp.dot(x, w[0], preferred_element_type=jnp.float32)
              for w in (w0_ref, w1_ref, w2_ref, w3_ref)]
        o_ref[0] = jnp.concatenate(ys, axis=1)

    return pl.pallas_call(
        body,
        grid=(N // R, 4),
        in_specs=[pl.BlockSpec((R, ic), lambda i, p: (i, 0)),
                  pl.BlockSpec((1, ic, oc), lambda i, p: (rl(p), 0, 0)),
                  pl.BlockSpec((1, ic, oc), lambda i, p: (rl(p) + 1, 0, 0)),
                  pl.BlockSpec((1, ic, oc), lambda i, p: (rl(p) + 3, 0, 0)),
                  pl.BlockSpec((1, ic, oc), lambda i, p: (rl(p) + 4, 0, 0))],
        out_specs=pl.BlockSpec((1, R, 4 * oc), lambda i, p: (p, i, 0)),
        out_shape=jax.ShapeDtypeStruct((4, N, 4 * oc), jnp.float32),
    )(inp, W, W, W, W)


def _epi(acc, cnt, x, root, b, oc, concat_halves, scale):
    ic = x.shape[1]
    R = 1000
    aw = 128

    def body(a_ref, c_ref, x_ref, r_ref, b_ref, o_ref):
        c0 = c_ref[0, :, 0] + c_ref[1, :, 0]
        rcp = (1.0 / jnp.maximum(c0, 1.0))[:, None]
        if concat_halves:
            s = jnp.concatenate([a_ref[0], a_ref[1]], axis=1)
        else:
            s = (a_ref[0] + a_ref[1])[:, :oc]
        rt = jnp.dot(x_ref[...] * scale, r_ref[...],
                     preferred_element_type=jnp.float32)
        o_ref[...] = jnp.maximum(s * rcp + rt + b_ref[...], 0.0)

    return pl.pallas_call(
        body,
        grid=(N // R,),
        in_specs=[pl.BlockSpec((2, R, aw), lambda i: (0, i, 0)),
                  pl.BlockSpec((2, R, 128), lambda i: (0, i, 0)),
                  pl.BlockSpec((R, ic), lambda i: (i, 0)),
                  pl.BlockSpec((ic, oc), lambda i: (0, 0)),
                  pl.BlockSpec((1, oc), lambda i: (0, 0))],
        out_specs=pl.BlockSpec((R, oc), lambda i: (i, 0)),
        out_shape=jax.ShapeDtypeStruct((N, oc), jnp.float32),
    )(acc, cnt, x, root, b.reshape(1, oc))


def _final(h, batch2, fc1_w, fc1_b, fc2_w, fc2_b):
    def body(h_ref, b_ref, w1_ref, b1_ref, w2_ref, b2_ref, o_ref):
        bt = b_ref[...]
        gids = lax.broadcasted_iota(jnp.int32, (1, NG), 1)
        mask = (bt == gids).astype(jnp.float32)
        gs = lax.dot_general(mask, h_ref[...], (((0,), (0,)), ((), ())),
                             preferred_element_type=jnp.float32)
        gc = jnp.sum(mask, axis=0)[:, None]
        g = gs / jnp.maximum(gc, 1.0)
        t = jnp.maximum(
            jnp.dot(g, w1_ref[...], preferred_element_type=jnp.float32)
            + b1_ref[...], 0.0)
        o_ref[...] = (jnp.dot(t, w2_ref[...], preferred_element_type=jnp.float32)
                      + b2_ref[...])

    return pl.pallas_call(
        body,
        out_shape=jax.ShapeDtypeStruct((NG, 16), jnp.float32),
    )(h, batch2, fc1_w, fc1_b.reshape(1, -1), fc2_w, fc2_b.reshape(1, -1))


def _sc_layer(table, sidx, dst, basis, O, col_split, quad):
    mesh = plsc.VectorSubcoreMesh(core_axis_name="c", subcore_axis_name="s")
    G = O // 16
    C = 64 if quad else 32
    rpe = 1 if quad else 2
    rw = 128 if quad else 2 * O
    epw = E_PAD // 16 if col_split else E_PAD // 32
    nch = epw // C
    nck = rpe * C

    @functools.partial(
        pl.kernel, mesh=mesh,
        out_type=jax.ShapeDtypeStruct((2, N_ACC, 128), jnp.float32),
        scratch_types=[
            [pltpu.VMEM((nck,), jnp.int32)] * 2,
            [pltpu.VMEM((nck,), jnp.int32)] * 2,
            [pltpu.VMEM((C,), jnp.int32)] * 2,
            [pltpu.VMEM((4 * C,), jnp.float32)] * 2,
            [pltpu.VMEM((nck, rw), jnp.float32)] * 2,
            pltpu.VMEM((C, 128), jnp.float32),
            pltpu.VMEM_SHARED((N_ACC, 128), jnp.float32),
            [pltpu.SemaphoreType.DMA] * 2,
            [pltpu.SemaphoreType.DMA] * 2,
        ])
    def k(table_h, idx_h, dst_h, basis_h, out_h,
          idx_v, idx2_v, dst_v, basis_v, rows_v, msg_v, acc, esem, gsem):
        c = lax.axis_index("c")
        s = lax.axis_index("s")

        def zb(i, _):
            for g in range(8):
                msg_v[i, pl.ds(g * 16, 16)] = jnp.zeros((16,), jnp.float32)
            return 0
        lax.fori_loop(0, C, zb, 0)
        r0 = s * RSUB
        for q in range(RSUB // C):
            pltpu.sync_copy(msg_v, acc.at[pl.ds(r0 + q * C, C)])
        plsc.subcore_barrier()

        if col_split:
            base0 = s * epw
            off = c * (NPAIR * N)
        else:
            base0 = (s * 2 + c) * epw

        def estream_issue(g, b):
            base = base0 + g * C
            pltpu.async_copy(idx_h.at[pl.ds(base * rpe, nck)], idx_v[b], esem[b])
            pltpu.async_copy(dst_h.at[pl.ds(base, C)], dst_v[b], esem[b])
            pltpu.async_copy(basis_h.at[pl.ds(base * 4, 4 * C)], basis_v[b], esem[b])

        def estream_wait(b):
            pltpu.make_async_copy(idx_h.at[pl.ds(0, nck)], idx_v[b], esem[b]).wait()
            pltpu.make_async_copy(dst_h.at[pl.ds(0, C)], dst_v[b], esem[b]).wait()
            pltpu.make_async_copy(basis_h.at[pl.ds(0, 4 * C)], basis_v[b], esem[b]).wait()

        def gather_issue(b):
            if col_split:
                for g in range(nck // 16):
                    sl = pl.ds(g * 16, 16)
                    idx2_v[b][sl] = idx_v[b][sl] + off
                iref = idx2_v[b]
            else:
                iref = idx_v[b]
            pltpu.async_copy(table_h.at[iref], rows_v[b], gsem[b])

        def gather_wait(b):
            pltpu.make_async_copy(table_h.at[idx_v[b]], rows_v[b], gsem[b]).wait()

        def combine_scatter(b):
            rows = rows_v[b]
            bas = basis_v[b]

            def edge4(t, _):
                bv = bas[pl.ds(t * 16, 16)]
                for q in range(4):
                    e = t * 4 + q
                    for g in range(G):
                        sl = pl.ds(g * 16, 16)
                        if quad:
                            m = (bv[4 * q] * rows[e, pl.ds(g * 16, 16)]
                                 + bv[4 * q + 1] * rows[e, pl.ds(O + g * 16, 16)]
                                 + bv[4 * q + 2] * rows[e, pl.ds(2 * O + g * 16, 16)]
                                 + bv[4 * q + 3] * rows[e, pl.ds(3 * O + g * 16, 16)])
                        else:
                            m = (bv[4 * q] * rows[2 * e, sl]
                                 + bv[4 * q + 1] * rows[2 * e, pl.ds(O + g * 16, 16)]
                                 + bv[4 * q + 2] * rows[2 * e + 1, sl]
                                 + bv[4 * q + 3] * rows[2 * e + 1, pl.ds(O + g * 16, 16)])
                        msg_v[e, sl] = m
                return 0
            lax.fori_loop(0, C // 4, edge4, 0)
            pltpu.sync_copy(msg_v, acc.at[dst_v[b]], add=True)

        estream_issue(0, 0)
        estream_issue(1, 1)
        estream_wait(0)
        gather_issue(0)

        def body(t, _):
            g0 = 2 * t
            for b in (0, 1):
                gather_wait(b)
                estream_wait(1 - b)
                gather_issue(1 - b)
                combine_scatter(b)
                estream_issue(g0 + b + 2, b)
            return 0
        lax.fori_loop(0, nch // 2 - 1, body, 0)
        gather_wait(0)
        estream_wait(1)
        gather_issue(1)
        combine_scatter(0)
        gather_wait(1)
        combine_scatter(1)

        plsc.subcore_barrier()
        for q in range(5):
            pltpu.sync_copy(acc.at[pl.ds(r0 + q * RQ, RQ)],
                            out_h.at[c, pl.ds(r0 + q * RQ, RQ)])

    return k(table, sidx, dst, basis)


def _sc_counts(dst, wvec):
    mesh = plsc.VectorSubcoreMesh(core_axis_name="c", subcore_axis_name="s")
    C = 64
    epw = E_PAD // 32
    nch = epw // C

    @functools.partial(
        pl.kernel, mesh=mesh,
        out_type=jax.ShapeDtypeStruct((2, N_ACC, 128), jnp.float32),
        scratch_types=[
            pltpu.VMEM((C,), jnp.int32),
            pltpu.VMEM((C,), jnp.float32),
            pltpu.VMEM((C, 128), jnp.float32),
            pltpu.VMEM_SHARED((N_ACC, 128), jnp.float32),
        ])
    def k(dst_h, w_h, out_h, dst_v, w_v, ones_v, acc):
        c = lax.axis_index("c")
        s = lax.axis_index("s")

        def zb(i, _):
            for g in range(8):
                ones_v[i, pl.ds(g * 16, 16)] = jnp.zeros((16,), jnp.float32)
            return 0
        lax.fori_loop(0, C, zb, 0)
        r0 = s * RSUB
        for q in range(RSUB // C):
            pltpu.sync_copy(ones_v, acc.at[pl.ds(r0 + q * C, C)])
        plsc.subcore_barrier()

        base0 = (s * 2 + c) * epw

        def chunk(i, _):
            base = base0 + i * C
            pltpu.sync_copy(dst_h.at[pl.ds(base, C)], dst_v)
            pltpu.sync_copy(w_h.at[pl.ds(base, C)], w_v)

            def edge16(t, _):
                wv16 = w_v[pl.ds(t * 16, 16)]
                for q in range(16):
                    w = wv16[q]
                    for g in range(8):
                        ones_v[t * 16 + q, pl.ds(g * 16, 16)] = (
                            w * jnp.ones((16,), jnp.float32))
                return 0
            lax.fori_loop(0, C // 16, edge16, 0)
            pltpu.sync_copy(ones_v, acc.at[dst_v], add=True)
            return 0
        lax.fori_loop(0, nch, chunk, 0)
        plsc.subcore_barrier()
        for q in range(5):
            pltpu.sync_copy(acc.at[pl.ds(r0 + q * RQ, RQ)],
                            out_h.at[c, pl.ds(r0 + q * RQ, RQ)])

    return k(dst, wvec)


def kernel(x, edge_index, edge_attr, batch, W0, root0, b0, W1, root1, b1,
           W2, root2, b2, W3, root3, b3, fc1_w, fc1_b, fc2_w, fc2_b):
    src = edge_index[0]
    dst = edge_index[1]
    pad = E_PAD - E
    srcp = jnp.concatenate([src, jnp.zeros((pad,), jnp.int32)])
    dstp = jnp.concatenate([dst, jnp.zeros((pad,), jnp.int32)])
    ea = jnp.concatenate([edge_attr, jnp.zeros((pad, 2), jnp.float32)], axis=0)
    wv = jnp.concatenate([jnp.ones((E,), jnp.float32),
                          jnp.zeros((pad,), jnp.float32)])
    rows = E_PAD // 128
    basis4, pidx2 = _prep(ea[:, 0].reshape(rows, 128),
                          ea[:, 1].reshape(rows, 128),
                          srcp.reshape(rows, 128), wv.reshape(rows, 128))
    basis_f = basis4.reshape(4, E_PAD).T.reshape(-1)
    pair_f = pidx2.reshape(2, E_PAD).T.reshape(-1)
    quad_f = pidx2.reshape(2, E_PAD)[0]

    cnt = _sc_counts(dstp, wv)

    table0 = _dense_pair(x, W0, 256, 1.0, True).reshape(2 * NPAIR * N, 256)
    acc0 = _sc_layer(table0, pair_f, dstp, basis_f, 128, True, False)
    h0 = _epi(acc0, cnt, x, root0, b0, 256, True, 1.0)

    table1 = _dense_pair(h0, W1, 128, 2.0, False).reshape(NPAIR * N, 256)
    acc1 = _sc_layer(table1, pair_f, dstp, basis_f, 128, False, False)
    h1 = _epi(acc1, cnt, h0, root1, b1, 128, False, 2.0)

    table2 = _dense_pair(h1, W2, 64, 2.0, False).reshape(NPAIR * N, 128)
    acc2 = _sc_layer(table2, pair_f, dstp, basis_f, 64, False, False)
    h2 = _epi(acc2, cnt, h1, root2, b2, 64, False, 2.0)

    table3 = _dense_quad(h2, W3, 2.0).reshape(4 * N, 128)
    acc3 = _sc_layer(table3, quad_f, dstp, basis_f, 32, False, True)
    h3 = _epi(acc3, cnt, h2, root3, b3, 32, False, 2.0)

    return _final(h3, batch.reshape(N, 1), fc1_w, fc1_b, fc2_w, fc2_b)


# speedup vs baseline: 1.4887x; 1.0385x over previous
import functools

import jax
import jax.numpy as jnp
from jax import lax
from jax.experimental import pallas as pl
from jax.experimental.pallas import tpu as pltpu
from jax.experimental.pallas import tpu_sc as plsc

N = 10000
E = 160000
KS = 3
NG = 8
NSLOT = 9
NPAIR = 6
E_PAD = 163840
N_ACC = 10240
RSUB = N_ACC // 16
RQ = 128


def _prep_body(ea0_ref, ea1_ref, src_ref, wv_ref, basis_ref, pidx_ref):
    p0 = ea0_ref[...] * (KS - 1.0)
    p1 = ea1_ref[...] * (KS - 1.0)
    i0 = jnp.clip(jnp.floor(p0), 0.0, KS - 2.0)
    i1 = jnp.clip(jnp.floor(p1), 0.0, KS - 2.0)
    f0 = p0 - i0
    f1 = p1 - i1
    i0i = i0.astype(jnp.int32)
    i1i = i1.astype(jnp.int32)
    wv = wv_ref[...]
    src = src_ref[...]
    j = 0
    for b0 in (0, 1):
        w0 = f0 if b0 else 1.0 - f0
        for b1 in (0, 1):
            w1 = f1 if b1 else 1.0 - f1
            basis_ref[j] = w0 * w1 * wv
            j += 1
    pr = (2 * i0i + i1i) * N + src
    pidx_ref[0] = pr
    pidx_ref[1] = pr + 2 * N


def _prep(ea0, ea1, src, wv):
    rows = E_PAD // 128
    return pl.pallas_call(
        _prep_body,
        out_shape=[jax.ShapeDtypeStruct((4, rows, 128), jnp.float32),
                   jax.ShapeDtypeStruct((2, rows, 128), jnp.int32)],
    )(ea0, ea1, src, wv)


def _dense_pair(inp, W, oc, scale, split):
    ic = inp.shape[1]
    R = 1000

    def rl(p):
        return 3 * (p // 2) + p % 2

    if split:
        out_shape = jax.ShapeDtypeStruct((2, NPAIR, N, 256), jnp.float32)
        out_spec = pl.BlockSpec((2, 1, R, 256), lambda i, p: (0, p, i, 0))
    else:
        out_shape = jax.ShapeDtypeStruct((NPAIR, N, 2 * oc), jnp.float32)
        out_spec = pl.BlockSpec((1, R, 2 * oc), lambda i, p: (p, i, 0))

    def body(x_ref, wl_ref, wr_ref, o_ref):
        x = x_ref[...] * scale
        yl = jnp.dot(x, wl_ref[0], preferred_element_type=jnp.float32)
        yr = jnp.dot(x, wr_ref[0], preferred_element_type=jnp.float32)
        if split:
            o_ref[0, 0] = jnp.concatenate([yl[:, :128], yr[:, :128]], axis=1)
            o_ref[1, 0] = jnp.concatenate([yl[:, 128:], yr[:, 128:]], axis=1)
        else:
            o_ref[0] = jnp.concatenate([yl, yr], axis=1)

    return pl.pallas_call(
        body,
        grid=(N // R, NPAIR),
        in_specs=[pl.BlockSpec((R, ic), lambda i, p: (i, 0)),
                  pl.BlockSpec((1, ic, oc), lambda i, p: (rl(p), 0, 0)),
                  pl.BlockSpec((1, ic, oc), lambda i, p: (rl(p) + 1, 0, 0))],
        out_specs=out_spec,
        out_shape=out_shape,
    )(inp, W, W)


def _dense_quad(inp, W, scale):
    ic = inp.shape[1]
    oc = W.shape[2]
    R = 1000

    def rl(p):
        return 3 * (p // 2) + p % 2

    def body(x_ref, w0_ref, w1_ref, w2_ref, w3_ref, o_ref):
        x = x_ref[...] * scale
        ys = [jnp.dot(x, w[0], preferred_element_type=jnp.float32)
              for w in (w0_ref, w1_ref, w2_ref, w3_ref)]
        o_ref[0] = jnp.concatenate(ys, axis=1)

    return pl.pallas_call(
        body,
        grid=(N // R, 4),
        in_specs=[pl.BlockSpec((R, ic), lambda i, p: (i, 0)),
                  pl.BlockSpec((1, ic, oc), lambda i, p: (rl(p), 0, 0)),
                  pl.BlockSpec((1, ic, oc), lambda i, p: (rl(p) + 1, 0, 0)),
                  pl.BlockSpec((1, ic, oc), lambda i, p: (rl(p) + 3, 0, 0)),
                  pl.BlockSpec((1, ic, oc), lambda i, p: (rl(p) + 4, 0, 0))],
        out_specs=pl.BlockSpec((1, R, 4 * oc), lambda i, p: (p, i, 0)),
        out_shape=jax.ShapeDtypeStruct((4, N, 4 * oc), jnp.float32),
    )(inp, W, W, W, W)


def _epi(acc, cnt, x, root, b, oc, concat_halves, scale):
    ic = x.shape[1]
    R = 1000
    aw = 128

    def body(a_ref, c_ref, x_ref, r_ref, b_ref, o_ref):
        c0 = c_ref[0, :, 0] + c_ref[1, :, 0]
        rcp = (1.0 / jnp.maximum(c0, 1.0))[:, None]
        if concat_halves:
            s = jnp.concatenate([a_ref[0], a_ref[1]], axis=1)
        else:
            s = (a_ref[0] + a_ref[1])[:, :oc]
        rt = jnp.dot(x_ref[...] * scale, r_ref[...],
                     preferred_element_type=jnp.float32)
        o_ref[...] = jnp.maximum(s * rcp + rt + b_ref[...], 0.0)

    return pl.pallas_call(
        body,
        grid=(N // R,),
        in_specs=[pl.BlockSpec((2, R, aw), lambda i: (0, i, 0)),
                  pl.BlockSpec((2, R, 128), lambda i: (0, i, 0)),
                  pl.BlockSpec((R, ic), lambda i: (i, 0)),
                  pl.BlockSpec((ic, oc), lambda i: (0, 0)),
                  pl.BlockSpec((1, oc), lambda i: (0, 0))],
        out_specs=pl.BlockSpec((R, oc), lambda i: (i, 0)),
        out_shape=jax.ShapeDtypeStruct((N, oc), jnp.float32),
    )(acc, cnt, x, root, b.reshape(1, oc))


def _final(h, batch2, fc1_w, fc1_b, fc2_w, fc2_b):
    def body(h_ref, b_ref, w1_ref, b1_ref, w2_ref, b2_ref, o_ref):
        bt = b_ref[...]
        gids = lax.broadcasted_iota(jnp.int32, (1, NG), 1)
        mask = (bt == gids).astype(jnp.float32)
        gs = lax.dot_general(mask, h_ref[...], (((0,), (0,)), ((), ())),
                             preferred_element_type=jnp.float32)
        gc = jnp.sum(mask, axis=0)[:, None]
        g = gs / jnp.maximum(gc, 1.0)
        t = jnp.maximum(
            jnp.dot(g, w1_ref[...], preferred_element_type=jnp.float32)
            + b1_ref[...], 0.0)
        o_ref[...] = (jnp.dot(t, w2_ref[...], preferred_element_type=jnp.float32)
                      + b2_ref[...])

    return pl.pallas_call(
        body,
        out_shape=jax.ShapeDtypeStruct((NG, 16), jnp.float32),
    )(h, batch2, fc1_w, fc1_b.reshape(1, -1), fc2_w, fc2_b.reshape(1, -1))


def _sc_layer(table, sidx, dst, basis, O, col_split, quad):
    mesh = plsc.VectorSubcoreMesh(core_axis_name="c", subcore_axis_name="s")
    G = O // 16
    C = 64 if quad else 32
    rpe = 1 if quad else 2
    rw = 128 if quad else 2 * O
    epw = E_PAD // 16 if col_split else E_PAD // 32
    nch = epw // C
    nck = rpe * C

    @functools.partial(
        pl.kernel, mesh=mesh,
        out_type=jax.ShapeDtypeStruct((2, N_ACC, 128), jnp.float32),
        scratch_types=[
            [pltpu.VMEM((nck,), jnp.int32)] * 2,
            [pltpu.VMEM((nck,), jnp.int32)] * 2,
            [pltpu.VMEM((C,), jnp.int32)] * 2,
            [pltpu.VMEM((4 * C,), jnp.float32)] * 2,
            [pltpu.VMEM((nck, rw), jnp.float32)] * 2,
            [pltpu.VMEM((C, 128), jnp.float32)] * 2,
            [pltpu.VMEM((C,), jnp.int32)] * 2,
            pltpu.VMEM_SHARED((N_ACC, 128), jnp.float32),
            [pltpu.SemaphoreType.DMA] * 2,
            [pltpu.SemaphoreType.DMA] * 2,
            [pltpu.SemaphoreType.DMA] * 2,
        ])
    def k(table_h, idx_h, dst_h, basis_h, out_h,
          idx_v, idx2_v, dst_v, basis_v, rows_v, msg_v, dsts_v, acc, esem,
          gsem, ssem):
        c = lax.axis_index("c")
        s = lax.axis_index("s")

        def zb(i, _):
            for g in range(8):
                msg_v[0][i, pl.ds(g * 16, 16)] = jnp.zeros((16,), jnp.float32)
                msg_v[1][i, pl.ds(g * 16, 16)] = jnp.zeros((16,), jnp.float32)
            return 0
        lax.fori_loop(0, C, zb, 0)
        r0 = s * RSUB
        for q in range(RSUB // C):
            pltpu.sync_copy(msg_v[0], acc.at[pl.ds(r0 + q * C, C)])
        plsc.subcore_barrier()

        if col_split:
            base0 = s * epw
            off = c * (NPAIR * N)
        else:
            base0 = (s * 2 + c) * epw

        def estream_issue(g, b):
            base = base0 + g * C
            pltpu.async_copy(idx_h.at[pl.ds(base * rpe, nck)], idx_v[b], esem[b])
            pltpu.async_copy(dst_h.at[pl.ds(base, C)], dst_v[b], esem[b])
            pltpu.async_copy(basis_h.at[pl.ds(base * 4, 4 * C)], basis_v[b], esem[b])

        def estream_wait(b):
            pltpu.make_async_copy(idx_h.at[pl.ds(0, nck)], idx_v[b], esem[b]).wait()
            pltpu.make_async_copy(dst_h.at[pl.ds(0, C)], dst_v[b], esem[b]).wait()
            pltpu.make_async_copy(basis_h.at[pl.ds(0, 4 * C)], basis_v[b], esem[b]).wait()

        def gather_issue(b):
            if col_split:
                for g in range(nck // 16):
                    sl = pl.ds(g * 16, 16)
                    idx2_v[b][sl] = idx_v[b][sl] + off
                iref = idx2_v[b]
            else:
                iref = idx_v[b]
            pltpu.async_copy(table_h.at[iref], rows_v[b], gsem[b])

        def gather_wait(b):
            pltpu.make_async_copy(table_h.at[idx_v[b]], rows_v[b], gsem[b]).wait()

        def scatter_wait(b):
            pltpu.make_async_copy(msg_v[b], acc.at[dsts_v[b]], ssem[b]).wait()

        def combine_scatter(b):
            rows = rows_v[b]
            bas = basis_v[b]

            def edge4(t, _):
                bv = bas[pl.ds(t * 16, 16)]
                for q in range(4):
                    e = t * 4 + q
                    for g in range(G):
                        sl = pl.ds(g * 16, 16)
                        if quad:
                            m = (bv[4 * q] * rows[e, pl.ds(g * 16, 16)]
                                 + bv[4 * q + 1] * rows[e, pl.ds(O + g * 16, 16)]
                                 + bv[4 * q + 2] * rows[e, pl.ds(2 * O + g * 16, 16)]
                                 + bv[4 * q + 3] * rows[e, pl.ds(3 * O + g * 16, 16)])
                        else:
                            m = (bv[4 * q] * rows[2 * e, sl]
                                 + bv[4 * q + 1] * rows[2 * e, pl.ds(O + g * 16, 16)]
                                 + bv[4 * q + 2] * rows[2 * e + 1, sl]
                                 + bv[4 * q + 3] * rows[2 * e + 1, pl.ds(O + g * 16, 16)])
                        msg_v[b][e, sl] = m
                return 0
            lax.fori_loop(0, C // 4, edge4, 0)
            for g in range(C // 16):
                sl = pl.ds(g * 16, 16)
                dsts_v[b][sl] = dst_v[b][sl]
            pltpu.async_copy(msg_v[b], acc.at[dsts_v[b]], ssem[b], add=True)

        estream_issue(0, 0)
        estream_issue(1, 1)
        estream_wait(0)
        gather_issue(0)
        for b in (0, 1):
            gather_wait(b)
            estream_wait(1 - b)
            gather_issue(1 - b)
            combine_scatter(b)
            estream_issue(b + 2, b)

        def body(t, _):
            g0 = 2 * t
            for b in (0, 1):
                gather_wait(b)
                estream_wait(1 - b)
                gather_issue(1 - b)
                scatter_wait(b)
                combine_scatter(b)
                estream_issue(g0 + b + 2, b)
            return 0
        lax.fori_loop(1, nch // 2 - 1, body, 0)
        gather_wait(0)
        estream_wait(1)
        gather_issue(1)
        scatter_wait(0)
        combine_scatter(0)
        gather_wait(1)
        scatter_wait(1)
        combine_scatter(1)
        scatter_wait(0)
        scatter_wait(1)

        plsc.subcore_barrier()
        for q in range(5):
            pltpu.sync_copy(acc.at[pl.ds(r0 + q * RQ, RQ)],
                            out_h.at[c, pl.ds(r0 + q * RQ, RQ)])

    return k(table, sidx, dst, basis)


def _sc_counts(dst, wvec):
    mesh = plsc.VectorSubcoreMesh(core_axis_name="c", subcore_axis_name="s")
    C = 64
    epw = E_PAD // 32
    nch = epw // C

    @functools.partial(
        pl.kernel, mesh=mesh,
        out_type=jax.ShapeDtypeStruct((2, N_ACC, 128), jnp.float32),
        scratch_types=[
            pltpu.VMEM((C,), jnp.int32),
            pltpu.VMEM((C,), jnp.float32),
            pltpu.VMEM((C, 128), jnp.float32),
            pltpu.VMEM_SHARED((N_ACC, 128), jnp.float32),
        ])
    def k(dst_h, w_h, out_h, dst_v, w_v, ones_v, acc):
        c = lax.axis_index("c")
        s = lax.axis_index("s")

        def zb(i, _):
            for g in range(8):
                ones_v[i, pl.ds(g * 16, 16)] = jnp.zeros((16,), jnp.float32)
            return 0
        lax.fori_loop(0, C, zb, 0)
        r0 = s * RSUB
        for q in range(RSUB // C):
            pltpu.sync_copy(ones_v, acc.at[pl.ds(r0 + q * C, C)])
        plsc.subcore_barrier()

        base0 = (s * 2 + c) * epw

        def chunk(i, _):
            base = base0 + i * C
            pltpu.sync_copy(dst_h.at[pl.ds(base, C)], dst_v)
            pltpu.sync_copy(w_h.at[pl.ds(base, C)], w_v)

            def edge16(t, _):
                wv16 = w_v[pl.ds(t * 16, 16)]
                for q in range(16):
                    w = wv16[q]
                    for g in range(8):
                        ones_v[t * 16 + q, pl.ds(g * 16, 16)] = (
                            w * jnp.ones((16,), jnp.float32))
                return 0
            lax.fori_loop(0, C // 16, edge16, 0)
            pltpu.sync_copy(ones_v, acc.at[dst_v], add=True)
            return 0
        lax.fori_loop(0, nch, chunk, 0)
        plsc.subcore_barrier()
        for q in range(5):
            pltpu.sync_copy(acc.at[pl.ds(r0 + q * RQ, RQ)],
                            out_h.at[c, pl.ds(r0 + q * RQ, RQ)])

    return k(dst, wvec)


def kernel(x, edge_index, edge_attr, batch, W0, root0, b0, W1, root1, b1,
           W2, root2, b2, W3, root3, b3, fc1_w, fc1_b, fc2_w, fc2_b):
    src = edge_index[0]
    dst = edge_index[1]
    pad = E_PAD - E
    srcp = jnp.concatenate([src, jnp.zeros((pad,), jnp.int32)])
    dstp = jnp.concatenate([dst, jnp.zeros((pad,), jnp.int32)])
    ea = jnp.concatenate([edge_attr, jnp.zeros((pad, 2), jnp.float32)], axis=0)
    wv = jnp.concatenate([jnp.ones((E,), jnp.float32),
                          jnp.zeros((pad,), jnp.float32)])
    rows = E_PAD // 128
    basis4, pidx2 = _prep(ea[:, 0].reshape(rows, 128),
                          ea[:, 1].reshape(rows, 128),
                          srcp.reshape(rows, 128), wv.reshape(rows, 128))
    basis_f = basis4.reshape(4, E_PAD).T.reshape(-1)
    pair_f = pidx2.reshape(2, E_PAD).T.reshape(-1)
    quad_f = pidx2.reshape(2, E_PAD)[0]

    cnt = _sc_counts(dstp, wv)

    table0 = _dense_pair(x, W0, 256, 1.0, True).reshape(2 * NPAIR * N, 256)
    acc0 = _sc_layer(table0, pair_f, dstp, basis_f, 128, True, False)
    h0 = _epi(acc0, cnt, x, root0, b0, 256, True, 1.0)

    table1 = _dense_pair(h0, W1, 128, 2.0, False).reshape(NPAIR * N, 256)
    acc1 = _sc_layer(table1, pair_f, dstp, basis_f, 128, False, False)
    h1 = _epi(acc1, cnt, h0, root1, b1, 128, False, 2.0)

    table2 = _dense_pair(h1, W2, 64, 2.0, False).reshape(NPAIR * N, 128)
    acc2 = _sc_layer(table2, pair_f, dstp, basis_f, 64, False, False)
    h2 = _epi(acc2, cnt, h1, root2, b2, 64, False, 2.0)

    table3 = _dense_quad(h2, W3, 2.0).reshape(4 * N, 128)
    acc3 = _sc_layer(table3, quad_f, dstp, basis_f, 32, False, True)
    h3 = _epi(acc3, cnt, h2, root3, b3, 32, False, 2.0)

    return _final(h3, batch.reshape(N, 1), fc1_w, fc1_b, fc2_w, fc2_b)
